# Initial kernel scaffold; baseline (speedup 1.0000x reference)
import jax
import jax.numpy as jnp
from jax.experimental import pallas as pl


def kernel(x, edge_index, batch, W0, b0, W1, b1, W2, b2, bias):
    raise NotImplementedError("write your pallas kernel here")


# speedup vs baseline: 4.1770x; 4.1770x over previous
import math

import jax
import jax.numpy as jnp
from jax import lax
from jax.experimental import pallas as pl
from jax.experimental.pallas import tpu as pltpu
from jax.experimental.pallas import tpu_sc as plsc

G = 512
NC = 2
NS = 16
CHUNK = 128
NBUF = 2
BN = 80


def _make_sc_agg(Dh, CH, Npad):
    RPT = Npad // NS
    mesh = plsc.VectorSubcoreMesh(core_axis_name="c", subcore_axis_name="s")

    def body(table, srcg, dstg, zrow, agg_out,
             is0, is1, id0, id1, rows0, rows1, acc,
             isem0, isem1, rsem0, rsem1):
        c = lax.axis_index("c")
        s = lax.axis_index("s")
        pltpu.sync_copy(zrow, acc.at[pl.ds(s * RPT, RPT)])
        plsc.subcore_barrier()

        isb = (is0, is1)
        idb = (id0, id1)
        rows = (rows0, rows1)
        isem = (isem0, isem1)
        rsem = (rsem0, rsem1)

        def fetch_idx(ch, b):
            pltpu.async_copy(srcg.at[c, s, ch], isb[b], isem[b])
            pltpu.async_copy(dstg.at[c, s, ch], idb[b], isem[b])

        def wait_idx(b):
            pltpu.make_async_copy(srcg.at[c, s, 0], isb[b], isem[b]).wait()
            pltpu.make_async_copy(dstg.at[c, s, 0], idb[b], isem[b]).wait()

        fetch_idx(0, 0)
        fetch_idx(1, 1)
        wait_idx(0)
        pltpu.async_copy(table.at[is0], rows0, rsem0)

        def step(ch, b):
            nb = 1 - b

            @pl.when(ch + 1 < CH)
            def _():
                wait_idx(nb)
                pltpu.async_copy(table.at[isb[nb]], rows[nb], rsem[nb])

            pltpu.make_async_copy(table.at[isb[b]], rows[b], rsem[b]).wait()
            pltpu.sync_copy(rows[b], acc.at[idb[b]], add=True)

            @pl.when(ch + 2 < CH)
            def _():
                fetch_idx(ch + 2, b)

        def outer(g, carry):
            step(g * 2, 0)
            step(g * 2 + 1, 1)
            return carry

        lax.fori_loop(0, CH // 2, outer, 0)
        plsc.subcore_barrier()
        pltpu.sync_copy(acc.at[pl.ds(s * RPT, RPT)],
                        agg_out.at[c, pl.ds(s * RPT, RPT)])

    return pl.kernel(
        body,
        out_type=jax.ShapeDtypeStruct((NC, Npad, Dh), jnp.float32),
        mesh=mesh,
        scratch_types=[
            pltpu.VMEM((CHUNK,), jnp.int32),
            pltpu.VMEM((CHUNK,), jnp.int32),
            pltpu.VMEM((CHUNK,), jnp.int32),
            pltpu.VMEM((CHUNK,), jnp.int32),
            pltpu.VMEM((CHUNK, Dh), jnp.float32),
            pltpu.VMEM((CHUNK, Dh), jnp.float32),
            pltpu.VMEM_SHARED((Npad, Dh), jnp.float32),
            pltpu.SemaphoreType.DMA,
            pltpu.SemaphoreType.DMA,
            pltpu.SemaphoreType.DMA,
            pltpu.SemaphoreType.DMA,
        ],
    )


def _tc_layer1(N, D, TH, Hc, Npad):
    nb = N // BN

    def body(x_ref, alo, ahi, w, bvec, batch_r, bchunk, h_out, r_out):
        i = pl.program_id(0)
        hin = x_ref[...] + alo[0] + ahi[0]
        h = jnp.maximum(
            jnp.dot(hin, w[...], preferred_element_type=jnp.float32)
            + bvec[...], 0.0)
        h_out[0] = h[:, :D]
        h_out[1] = h[:, D:]
        bb = batch_r[0, 0, :]
        oh = (lax.broadcasted_iota(jnp.int32, (G, BN), 0)
              == bb[None, :]).astype(jnp.float32)

        @pl.when(i == 0)
        def _():
            r_out[...] = jnp.concatenate(
                [jnp.broadcast_to(bchunk[...], (G, bchunk.shape[1])),
                 jnp.zeros((G, TH - bchunk.shape[1]), jnp.float32)], axis=1)

        r_out[...] += jnp.dot(oh, h, preferred_element_type=jnp.float32)

    return pl.pallas_call(
        body,
        grid=(nb,),
        in_specs=[
            pl.BlockSpec((BN, D), lambda i: (i, 0)),
            pl.BlockSpec((1, BN, D), lambda i: (0, i, 0)),
            pl.BlockSpec((1, BN, D), lambda i: (1, i, 0)),
            pl.BlockSpec((D, TH), lambda i: (0, 0)),
            pl.BlockSpec((1, TH), lambda i: (0, 0)),
            pl.BlockSpec((1, 1, BN), lambda i: (i, 0, 0)),
            pl.BlockSpec((1, Hc), lambda i: (0, 0)),
        ],
        out_specs=[
            pl.BlockSpec((2, BN, D), lambda i: (0, i, 0)),
            pl.BlockSpec((G, TH), lambda i: (0, 0)),
        ],
        out_shape=[
            jax.ShapeDtypeStruct((2, N, D), jnp.float32),
            jax.ShapeDtypeStruct((G, TH), jnp.float32),
        ],
    )


def _tc_layer23(N, D, TH, Hc, Npad, write_h):
    nb = N // BN

    def body(hlo, hhi, alo, ahi, w, bvec, batch_r, bchunk, *outs):
        i = pl.program_id(0)
        if write_h:
            h_out, r_out = outs
        else:
            (r_out,) = outs
        hin = (jnp.concatenate([hlo[0], hhi[0]], axis=1)
               + jnp.concatenate([alo[0], ahi[0]], axis=1))
        h = jnp.maximum(
            jnp.dot(hin, w[...], preferred_element_type=jnp.float32)
            + bvec[...], 0.0)
        if write_h:
            h_out[0] = h[:, :D]
            h_out[1] = h[:, D:]
        bb = batch_r[0, 0, :]
        oh = (lax.broadcasted_iota(jnp.int32, (G, BN), 0)
              == bb[None, :]).astype(jnp.float32)

        @pl.when(i == 0)
        def _():
            r_out[...] = jnp.concatenate(
                [jnp.broadcast_to(bchunk[...], (G, bchunk.shape[1])),
                 jnp.zeros((G, TH - bchunk.shape[1]), jnp.float32)], axis=1)

        r_out[...] += jnp.dot(oh, h, preferred_element_type=jnp.float32)

    out_specs = [pl.BlockSpec((G, TH), lambda i: (0, 0))]
    out_shape = [jax.ShapeDtypeStruct((G, TH), jnp.float32)]
    if write_h:
        out_specs = [pl.BlockSpec((2, BN, D), lambda i: (0, i, 0))] + out_specs
        out_shape = [jax.ShapeDtypeStruct((2, N, D), jnp.float32)] + out_shape

    return pl.pallas_call(
        body,
        grid=(nb,),
        in_specs=[
            pl.BlockSpec((1, BN, D), lambda i: (0, i, 0)),
            pl.BlockSpec((1, BN, D), lambda i: (1, i, 0)),
            pl.BlockSpec((1, BN, D), lambda i: (0, i, 0)),
            pl.BlockSpec((1, BN, D), lambda i: (1, i, 0)),
            pl.BlockSpec((TH, TH), lambda i: (0, 0)),
            pl.BlockSpec((1, TH), lambda i: (0, 0)),
            pl.BlockSpec((1, 1, BN), lambda i: (i, 0, 0)),
            pl.BlockSpec((1, Hc), lambda i: (0, 0)),
        ],
        out_specs=out_specs,
        out_shape=out_shape,
    )


def kernel(x, edge_index, batch, W0, b0, W1, b1, W2, b2, bias):
    N, D = x.shape
    E = edge_index.shape[1]
    T, _, H = W0.shape
    TH = T * H
    L = 3

    CH1 = -(-E // (NC * NS * CHUNK))
    CH1 += CH1 % NBUF
    Ep = NC * NS * CH1 * CHUNK
    CH2 = Ep // (NS * CHUNK)
    assert CH2 % NBUF == 0
    align = (BN * NS * 8) // math.gcd(BN, NS * 8)
    Npad = -(-(N + 1) // align) * align
    assert N % BN == 0

    src = edge_index[0]
    dst = edge_index[1]
    pad = Ep - E
    srcp = jnp.concatenate([src, jnp.zeros((pad,), jnp.int32)])
    dstp = jnp.concatenate([dst, jnp.full((pad,), N, jnp.int32)])
    src1g = srcp.reshape(NC, NS, CH1, CHUNK)
    dst1g = dstp.reshape(NC, NS, CH1, CHUNK)
    src2g = jnp.stack([srcp, srcp + N]).reshape(NC, NS, CH2, CHUNK)
    dst2g = jnp.broadcast_to(dstp, (NC, Ep)).reshape(NC, NS, CH2, CHUNK)

    W0c = W0.transpose(1, 0, 2).reshape(D, TH)
    b0c = b0.reshape(1, TH)
    W1bd = jax.scipy.linalg.block_diag(*[W1[t] for t in range(T)])
    W2bd = jax.scipy.linalg.block_diag(*[W2[t] for t in range(T)])
    b1c = b1.reshape(1, TH)
    b2c = b2.reshape(1, TH)
    batch3 = batch.reshape(N // BN, 1, BN)
    bias_c = [bias[:, 0, l * H:(l + 1) * H] for l in range(L)]

    zrow = jnp.zeros((Npad // NS, D), jnp.float32)

    agg_l1 = _make_sc_agg(D, CH1, Npad)
    agg_l23 = _make_sc_agg(D, CH2, Npad)
    tc1 = _tc_layer1(N, D, TH, H, Npad)
    tc2 = _tc_layer23(N, D, TH, H, Npad, write_h=True)
    tc3 = _tc_layer23(N, D, TH, H, Npad, write_h=False)

    agg1 = agg_l1(x, src1g, dst1g, zrow)
    h1, r1 = tc1(x, agg1, agg1, W0c, b0c, batch3, bias_c[0])
    agg2 = agg_l23(h1.reshape(2 * N, D), src2g, dst2g, zrow)
    h2, r2 = tc2(h1, h1, agg2, agg2, W1bd, b1c, batch3, bias_c[1])
    agg3 = agg_l23(h2.reshape(2 * N, D), src2g, dst2g, zrow)
    (r3,) = tc3(h2, h2, agg3, agg3, W2bd, b2c, batch3, bias_c[2])

    out = jnp.stack([r.reshape(G, T, H) for r in (r1, r2, r3)], axis=2)
    return out.reshape(G, T, L * H)
